# baseline (speedup 1.0000x reference)
import jax
import jax.numpy as jnp
from jax.experimental import pallas as pl
from jax.experimental.pallas import tpu as pltpu

_SEARCH = 5
_PRE = 7
_KNN = 9
_CIN = 5
_STEM = 32
_BH = 8
_BW = 128
_HALO_H = 2 * _BH
_HALO_W = _BW + 128
_CENTER = (_SEARCH * _SEARCH - 1) // 2
_NROWS = 48


def _tree_argmin(slots, d):
    nodes = [(d[s], None, s) for s in slots]
    while len(nodes) > 1:
        nxt = []
        for a in range(0, len(nodes) - 1, 2):
            vl, il, cl = nodes[a]
            vr, ir, cr = nodes[a + 1]
            lt = vr < vl
            v = jnp.where(lt, vr, vl)
            ilv = jnp.full_like(vl, jnp.float32(cl)) if il is None else il
            irv = jnp.full_like(vr, jnp.float32(cr)) if ir is None else ir
            nxt.append((v, jnp.where(lt, irv, ilv), None))
        if len(nodes) % 2:
            nxt.append(nodes[-1])
        nodes = nxt
    v, i, c = nodes[0]
    return jnp.full_like(v, jnp.float32(c)) if i is None else i


def _body(xp_ref, pxp_ref, w_ref, pw_ref, out_ref, pout_ref,
          gi_scr, gp_scr, d_scr, ai_scr):
    b = pl.program_id(0)
    hb = pl.program_id(1)
    wb = pl.program_id(2)
    h0 = hb * _BH
    w0 = wb * _BW

    def halo(ref, c):
        return ref[b, c, pl.ds(h0, _HALO_H), pl.ds(w0, _HALO_W)]

    def shifted_cols(hal, k, roff):
        return [jax.lax.slice(hal, (0, dj + roff), (_HALO_H, dj + roff + _BW))
                for dj in range(k)]

    def slot_val(shift, di, dj, roff):
        return jax.lax.slice(shift[dj], (di + roff, 0), (di + roff + _BH, _BW))

    zeros = jnp.zeros((_BH, _BW), jnp.float32)

    def diffs_to_scratch(src_ref, k, roff, center, skip):
        hal0 = halo(src_ref, 0)
        shift = shifted_cols(hal0, k, roff)
        for s in range(k * k):
            if s == skip:
                continue
            di, dj = divmod(s, k)
            d_scr[s] = jnp.abs(slot_val(shift, di, dj, roff) - center)

    def topk_to_scratch(slots, npass):
        for j in range(npass):
            d = {s: d_scr[s] for s in slots}
            ai = _tree_argmin(slots, d)
            ai_scr[j] = ai
            if j + 1 < npass:
                for s in slots:
                    d_scr[s] = jnp.where(ai == jnp.float32(s), jnp.float32(jnp.inf), d[s])

    def gather_to_scratch(src_ref, k, roff, npass, jbase, scr, skip):
        for c in range(_CIN):
            shift = shifted_cols(halo(src_ref, c), k, roff)
            if skip is not None:
                scr[c * _KNN] = slot_val(shift, 2, 2, roff)
            ais = [ai_scr[j] for j in range(npass)]
            acc = [None] * npass
            for s in range(k * k):
                if s == skip:
                    continue
                di, dj = divmod(s, k)
                v = slot_val(shift, di, dj, roff)
                for j in range(npass):
                    m = ais[j] == jnp.float32(s)
                    acc[j] = jnp.where(m, v, zeros if acc[j] is None else acc[j])
            for j in range(npass):
                scr[c * _KNN + jbase + j] = acc[j]

    def matmul_out(scr, wref, oref):
        scr[45] = zeros
        scr[46] = zeros
        scr[47] = zeros
        g = scr[...].reshape(_NROWS, _BH * _BW)
        o = jnp.maximum(jnp.dot(wref[...], g, preferred_element_type=jnp.float32), 0.0)
        oref[0] = o.reshape(_STEM, _BH, _BW)

    center = slot_val(shifted_cols(halo(xp_ref, 0), 1, 3), 0, 0, 3)
    s_slots = [s for s in range(_SEARCH * _SEARCH) if s != _CENTER]
    diffs_to_scratch(xp_ref, _SEARCH, 1, center, _CENTER)
    topk_to_scratch(s_slots, _KNN - 1)
    gather_to_scratch(xp_ref, _SEARCH, 1, _KNN - 1, 1, gi_scr, _CENTER)
    matmul_out(gi_scr, w_ref, out_ref)

    p_slots = list(range(_PRE * _PRE))
    diffs_to_scratch(pxp_ref, _PRE, 0, center, None)
    topk_to_scratch(p_slots, _KNN)
    gather_to_scratch(pxp_ref, _PRE, 0, _KNN, 0, gp_scr, None)

    ax = gi_scr[1 * _KNN]
    ay = gi_scr[2 * _KNN]
    az = gi_scr[3 * _KNN]
    for j in range(_KNN):
        x0 = gp_scr[1 * _KNN + j] - ax
        y0 = gp_scr[2 * _KNN + j] - ay
        z0 = gp_scr[3 * _KNN + j] - az
        xy = x0 * x0 + y0 * y0
        z2 = z0 * z0
        r = jnp.sqrt(xy + z2)
        t = jnp.arctan2(jnp.sqrt(xy), z2)
        gp_scr[1 * _KNN + j] = r
        gp_scr[2 * _KNN + j] = t
        gp_scr[3 * _KNN + j] = jnp.arctan2(t * t, r * r)
    matmul_out(gp_scr, pw_ref, pout_ref)


def kernel(x, pre_x, range_weight, pre_range_weight):
    B, C, H, W = x.shape
    pad = (_PRE - 1) // 2
    hpad2 = _HALO_H - pad
    wpad2 = _HALO_W + 128 - _BW - pad
    xp = jnp.pad(x, ((0, 0), (0, 0), (pad, hpad2), (pad, wpad2)))
    pxp = jnp.pad(pre_x, ((0, 0), (0, 0), (pad, hpad2), (pad, wpad2)))
    w1 = jnp.pad(range_weight.reshape(_STEM, _CIN * _KNN), ((0, 0), (0, _NROWS - _CIN * _KNN)))
    w2 = jnp.pad(pre_range_weight.reshape(_STEM, _CIN * _KNN), ((0, 0), (0, _NROWS - _CIN * _KNN)))

    grid = (B, H // _BH, W // _BW)
    out_sds = jax.ShapeDtypeStruct((B, _STEM, H, W), jnp.float32)
    in_specs = [
        pl.BlockSpec(xp.shape, lambda b, h, w: (0, 0, 0, 0)),
        pl.BlockSpec(pxp.shape, lambda b, h, w: (0, 0, 0, 0)),
        pl.BlockSpec(w1.shape, lambda b, h, w: (0, 0)),
        pl.BlockSpec(w2.shape, lambda b, h, w: (0, 0)),
    ]
    out_specs = [
        pl.BlockSpec((1, _STEM, _BH, _BW), lambda b, h, w: (b, 0, h, w)),
        pl.BlockSpec((1, _STEM, _BH, _BW), lambda b, h, w: (b, 0, h, w)),
    ]
    out, pre_out = pl.pallas_call(
        _body,
        grid=grid,
        in_specs=in_specs,
        out_specs=out_specs,
        out_shape=[out_sds, out_sds],
        scratch_shapes=[
            pltpu.VMEM((_NROWS, _BH, _BW), jnp.float32),
            pltpu.VMEM((_NROWS, _BH, _BW), jnp.float32),
            pltpu.VMEM((_PRE * _PRE, _BH, _BW), jnp.float32),
            pltpu.VMEM((_KNN, _BH, _BW), jnp.float32),
        ],
    )(xp, pxp, w1, w2)
    return (out, pre_out)


# speedup vs baseline: 1.0292x; 1.0292x over previous
import jax
import jax.numpy as jnp
from jax.experimental import pallas as pl
from jax.experimental.pallas import tpu as pltpu

_SEARCH = 5
_PRE = 7
_KNN = 9
_CIN = 5
_STEM = 32
_BH = 8
_BW = 128
_HALO_H = 2 * _BH
_HALO_W = _BW + 128
_CENTER = (_SEARCH * _SEARCH - 1) // 2
_NROWS = 48
_GROUPS = ((0, 1, 2), (3, 4))


def _tree_argmin(slots, d):
    nodes = [(d[s], None, s) for s in slots]
    while len(nodes) > 1:
        nxt = []
        for a in range(0, len(nodes) - 1, 2):
            vl, il, cl = nodes[a]
            vr, ir, cr = nodes[a + 1]
            lt = vr < vl
            v = jnp.where(lt, vr, vl)
            ilv = jnp.full_like(vl, jnp.float32(cl)) if il is None else il
            irv = jnp.full_like(vr, jnp.float32(cr)) if ir is None else ir
            nxt.append((v, jnp.where(lt, irv, ilv), None))
        if len(nodes) % 2:
            nxt.append(nodes[-1])
        nodes = nxt
    v, i, c = nodes[0]
    return jnp.full_like(v, jnp.float32(c)) if i is None else i


def _body(xp_ref, pxp_ref, w_ref, pw_ref, out_ref, pout_ref,
          gi_scr, gp_scr, d1_scr, d2_scr, a1_scr, a2_scr):
    b = pl.program_id(0)
    hb = pl.program_id(1)
    wb = pl.program_id(2)
    h0 = hb * _BH
    w0 = wb * _BW

    def halo(ref, c):
        return ref[b, c, pl.ds(h0, _HALO_H), pl.ds(w0, _HALO_W)]

    def col(hal, dj):
        return jax.lax.slice(hal, (0, dj), (_HALO_H, dj + _BW))

    def rows(shifted, di):
        return jax.lax.slice(shifted, (di, 0), (di + _BH, _BW))

    zeros = jnp.zeros((_BH, _BW), jnp.float32)

    def diffs_to_scratch(src_ref, k, roff, center, skip, d_scr):
        hal0 = halo(src_ref, 0)
        for dj in range(k):
            shifted = col(hal0, dj + roff)
            for di in range(k):
                s = di * k + dj
                if s == skip:
                    continue
                d_scr[s] = jnp.abs(rows(shifted, di + roff) - center)

    def topk_pass(slots, j, npass, d_scr, a_scr):
        d = {s: d_scr[s] for s in slots}
        ai = _tree_argmin(slots, d)
        a_scr[j] = ai
        if j + 1 < npass:
            for s in slots:
                d_scr[s] = jnp.where(ai == jnp.float32(s), jnp.float32(jnp.inf), d[s])

    def gather_to_scratch(src_ref, k, roff, npass, jbase, scr, skip, a_scr):
        for group in _GROUPS:
            hals = {c: halo(src_ref, c) for c in group}
            if skip is not None:
                for c in group:
                    scr[c * _KNN] = rows(col(hals[c], 2 + roff), 2 + roff)
            ais = [a_scr[j] for j in range(npass)]
            acc = {}
            for dj in range(k):
                shifted = {c: col(hals[c], dj + roff) for c in group}
                for di in range(k):
                    s = di * k + dj
                    if s == skip:
                        continue
                    vals = {c: rows(shifted[c], di + roff) for c in group}
                    for j in range(npass):
                        m = ais[j] == jnp.float32(s)
                        for c in group:
                            prev = acc.get((c, j))
                            acc[c, j] = jnp.where(m, vals[c], zeros if prev is None else prev)
            for (c, j), v in acc.items():
                scr[c * _KNN + jbase + j] = v

    def matmul_out(scr, wref, oref):
        scr[45] = zeros
        scr[46] = zeros
        scr[47] = zeros
        g = scr[...].reshape(_NROWS, _BH * _BW)
        o = jnp.maximum(jnp.dot(wref[...], g, preferred_element_type=jnp.float32), 0.0)
        oref[0] = o.reshape(_STEM, _BH, _BW)

    center = rows(col(halo(xp_ref, 0), 3), 3)
    s_slots = [s for s in range(_SEARCH * _SEARCH) if s != _CENTER]
    p_slots = list(range(_PRE * _PRE))
    diffs_to_scratch(xp_ref, _SEARCH, 1, center, _CENTER, d1_scr)
    diffs_to_scratch(pxp_ref, _PRE, 0, center, None, d2_scr)
    for j in range(_KNN):
        if j < _KNN - 1:
            topk_pass(s_slots, j, _KNN - 1, d1_scr, a1_scr)
        topk_pass(p_slots, j, _KNN, d2_scr, a2_scr)
    gather_to_scratch(xp_ref, _SEARCH, 1, _KNN - 1, 1, gi_scr, _CENTER, a1_scr)
    matmul_out(gi_scr, w_ref, out_ref)
    gather_to_scratch(pxp_ref, _PRE, 0, _KNN, 0, gp_scr, None, a2_scr)

    ax = gi_scr[1 * _KNN]
    ay = gi_scr[2 * _KNN]
    az = gi_scr[3 * _KNN]
    for j in range(_KNN):
        x0 = gp_scr[1 * _KNN + j] - ax
        y0 = gp_scr[2 * _KNN + j] - ay
        z0 = gp_scr[3 * _KNN + j] - az
        xy = x0 * x0 + y0 * y0
        z2 = z0 * z0
        r = jnp.sqrt(xy + z2)
        t = jnp.arctan2(jnp.sqrt(xy), z2)
        gp_scr[1 * _KNN + j] = r
        gp_scr[2 * _KNN + j] = t
        gp_scr[3 * _KNN + j] = jnp.arctan2(t * t, r * r)
    matmul_out(gp_scr, pw_ref, pout_ref)


def kernel(x, pre_x, range_weight, pre_range_weight):
    B, C, H, W = x.shape
    pad = (_PRE - 1) // 2
    hpad2 = _HALO_H - pad
    wpad2 = _HALO_W + 128 - _BW - pad
    xp = jnp.pad(x, ((0, 0), (0, 0), (pad, hpad2), (pad, wpad2)))
    pxp = jnp.pad(pre_x, ((0, 0), (0, 0), (pad, hpad2), (pad, wpad2)))
    w1 = jnp.pad(range_weight.reshape(_STEM, _CIN * _KNN), ((0, 0), (0, _NROWS - _CIN * _KNN)))
    w2 = jnp.pad(pre_range_weight.reshape(_STEM, _CIN * _KNN), ((0, 0), (0, _NROWS - _CIN * _KNN)))

    grid = (B, H // _BH, W // _BW)
    out_sds = jax.ShapeDtypeStruct((B, _STEM, H, W), jnp.float32)
    in_specs = [
        pl.BlockSpec(xp.shape, lambda b, h, w: (0, 0, 0, 0)),
        pl.BlockSpec(pxp.shape, lambda b, h, w: (0, 0, 0, 0)),
        pl.BlockSpec(w1.shape, lambda b, h, w: (0, 0)),
        pl.BlockSpec(w2.shape, lambda b, h, w: (0, 0)),
    ]
    out_specs = [
        pl.BlockSpec((1, _STEM, _BH, _BW), lambda b, h, w: (b, 0, h, w)),
        pl.BlockSpec((1, _STEM, _BH, _BW), lambda b, h, w: (b, 0, h, w)),
    ]
    out, pre_out = pl.pallas_call(
        _body,
        grid=grid,
        in_specs=in_specs,
        out_specs=out_specs,
        out_shape=[out_sds, out_sds],
        scratch_shapes=[
            pltpu.VMEM((_NROWS, _BH, _BW), jnp.float32),
            pltpu.VMEM((_NROWS, _BH, _BW), jnp.float32),
            pltpu.VMEM((_SEARCH * _SEARCH, _BH, _BW), jnp.float32),
            pltpu.VMEM((_PRE * _PRE, _BH, _BW), jnp.float32),
            pltpu.VMEM((_KNN, _BH, _BW), jnp.float32),
            pltpu.VMEM((_KNN, _BH, _BW), jnp.float32),
        ],
    )(xp, pxp, w1, w2)
    return (out, pre_out)
